# baseline (speedup 1.0000x reference)
import functools

import jax
import jax.numpy as jnp
from jax import lax
from jax.experimental import pallas as pl
from jax.experimental.pallas import tpu as pltpu
from jax.experimental.pallas import tpu_sc as plsc

NC = 2
NS = 16
NW = NC * NS
L = 16
NBUF = 2


def _node_mm2_body(h, aw, ab, bw, bb, a_o, b_o):
    x = h[...]
    a_o[...] = jnp.dot(x, aw[...], preferred_element_type=jnp.float32) + ab[...]
    b_o[...] = jnp.dot(x, bw[...], preferred_element_type=jnp.float32) + bb[...]


def _edge_mm_body(e, cw, cb, o):
    o[...] = jnp.dot(e[...], cw[...], preferred_element_type=jnp.float32) + cb[...]


def _eout_body(enew, ein, esc, esh, o):
    y = jnp.maximum(enew[...] * esc[...] + esh[...], 0.0)
    o[...] = ein[...] + y


def _stage_b_body(psum, bh, bns, bnq, gamma, beta, eee_o, esc_o, esh_o, *,
                  n_edges):
    n = bh.shape[0]
    ss = psum[:n, :] + psum[n:, :]
    eee_o[...] = bh[...] / (ss + 1e-6)
    sums = jnp.sum(bns[...], axis=0, keepdims=True)
    sumsq = jnp.sum(bnq[...], axis=0, keepdims=True)
    mean = sums / n_edges
    var = sumsq / n_edges - mean * mean
    scale = gamma[...] * lax.rsqrt(var + 1e-5)
    shift = beta[...] - mean * scale
    esc_o[...] = jnp.broadcast_to(scale, esc_o.shape)
    esh_o[...] = jnp.broadcast_to(shift, esh_o.shape)


def _stage_c_body(ah, psumh, h, gamma, beta, out):
    n = ah.shape[0]
    s = ah[...] + psumh[:n, :] + psumh[n:, :]
    mean = jnp.mean(s, axis=0, keepdims=True)
    d = s - mean
    var = jnp.mean(d * d, axis=0, keepdims=True)
    y = gamma[...] * d * lax.rsqrt(var + 1e-5) + beta[...]
    out[...] = h[...] + jnp.maximum(y, 0.0)


def _zero_fill(buf, rows):
    d = buf.shape[1]
    zero = jnp.zeros((L,), jnp.float32)

    def body(r, _):
        for g in range(d // L):
            buf[r, pl.ds(g * L, L)] = zero
        return 0

    lax.fori_loop(0, rows, body, 0, unroll=False)


def _spmem_zero(acc, zbuf, row0, rows):
    step = zbuf.shape[0]
    off = 0
    while off < rows:
        sz = min(step, rows - off)
        pltpu.sync_copy(zbuf.at[pl.ds(0, sz)],
                        acc.at[pl.ds(pl.multiple_of(row0 + off, 8), sz)])
        off += sz


def _spmem_dump(acc, stage, out_hbm, out_base, row0, rows):
    step = stage.shape[0]
    off = 0
    while off < rows:
        sz = min(step, rows - off)
        pltpu.sync_copy(acc.at[pl.ds(pl.multiple_of(row0 + off, 8), sz)],
                        stage.at[pl.ds(0, sz)])
        pltpu.sync_copy(stage.at[pl.ds(0, sz)],
                        out_hbm.at[pl.ds(
                            pl.multiple_of(out_base + row0 + off, 8), sz)])
        off += sz


def _row_split(n):
    rps = -(-n // NS)
    rps = -(-rps // 8) * 8
    last = n - rps * (NS - 1)
    assert 0 < last <= rps and rps % 8 == 0
    return rps, last


def _sigmoid(x):
    return 1.0 / (1.0 + jnp.exp(-x))


def _trip(base_chunks, extra):
    kmax = base_chunks + (1 if extra else 0) + 1
    return -(-kmax // NBUF) * NBUF


def _make_pass1(n, e_cnt, d, chunk):
    nch = e_cnt // chunk
    assert nch * chunk == e_cnt and chunk % 8 == 0
    base_chunks = nch // NW
    extra = nch % NW
    rps, rps_last = _row_split(n)
    grp = d // L
    kmax = _trip(base_chunks, extra)
    mesh = plsc.VectorSubcoreMesh(core_axis_name="c", subcore_axis_name="s",
                                  num_cores=NC, num_subcores=NS)

    @functools.partial(
        pl.kernel,
        out_type=[
            jax.ShapeDtypeStruct((e_cnt, d), jnp.float32),
            jax.ShapeDtypeStruct((NC * n, d), jnp.float32),
            jax.ShapeDtypeStruct((NW * d,), jnp.float32),
            jax.ShapeDtypeStruct((NW * d,), jnp.float32),
        ],
        mesh=mesh,
        scratch_types=(
            [pltpu.VMEM_SHARED((n, d), jnp.float32)]
            + [pltpu.VMEM((chunk // 2,), jnp.int32)] * (2 * NBUF)
            + [pltpu.VMEM((chunk // 2,), jnp.int32)] * (2 * NBUF)
            + [pltpu.VMEM((chunk // 2,), jnp.int32)] * (2 * NBUF)
            + [pltpu.VMEM((chunk, d), jnp.float32)] * NBUF
            + [pltpu.VMEM((chunk, d), jnp.float32)] * NBUF
            + [pltpu.VMEM((chunk, d), jnp.float32)] * NBUF
            + [pltpu.VMEM((d,), jnp.float32)] * 2
            + [pltpu.SemaphoreType.DMA((NBUF,))] * 7
        ),
    )
    def pass1(ce, dh, eh, src, dst, enew_o, psum_o, bns_o, bnq_o,
              acc, *rest):
        h2 = chunk // 2
        is_lo, is_hi = rest[0:NBUF], rest[NBUF:2 * NBUF]
        id_lo, id_hi = rest[2 * NBUF:3 * NBUF], rest[3 * NBUF:4 * NBUF]
        ic_lo, ic_hi = rest[4 * NBUF:5 * NBUF], rest[5 * NBUF:6 * NBUF]
        ceb = rest[6 * NBUF:7 * NBUF]
        dhb = rest[7 * NBUF:8 * NBUF]
        ehb = rest[8 * NBUF:9 * NBUF]
        bns_b, bnq_b = rest[9 * NBUF], rest[9 * NBUF + 1]
        sis, sid, sce, sdh, seh, soe, sos = rest[9 * NBUF + 2:]

        c = lax.axis_index("c")
        s = lax.axis_index("s")
        wid = s * NC + c

        _zero_fill(dhb[0], chunk)
        row0 = s * rps

        @pl.when(s < NS - 1)
        def _():
            _spmem_zero(acc, dhb[0], row0, rps)

        @pl.when(s == NS - 1)
        def _():
            _spmem_zero(acc, dhb[0], row0, rps_last)

        plsc.subcore_barrier()

        n_my = base_chunks + jnp.where(wid < extra, 1, 0)

        def idx_issue(k, b):
            base = (k * NW + wid) * chunk
            pltpu.async_copy(src.at[pl.ds(base, h2)], is_lo[b], sis.at[b])
            pltpu.async_copy(src.at[pl.ds(base + h2, h2)], is_hi[b], sis.at[b])
            pltpu.async_copy(dst.at[pl.ds(base, h2)], id_lo[b], sid.at[b])
            pltpu.async_copy(dst.at[pl.ds(base + h2, h2)], id_hi[b], sid.at[b])

        def idx_wait(b):
            pltpu.make_async_copy(src.at[pl.ds(0, h2)], is_lo[b],
                                  sis.at[b]).wait()
            pltpu.make_async_copy(src.at[pl.ds(0, h2)], is_hi[b],
                                  sis.at[b]).wait()
            pltpu.make_async_copy(dst.at[pl.ds(0, h2)], id_lo[b],
                                  sid.at[b]).wait()
            pltpu.make_async_copy(dst.at[pl.ds(0, h2)], id_hi[b],
                                  sid.at[b]).wait()

        def in_issue(k, b):
            base = (k * NW + wid) * chunk
            pltpu.async_copy(ce.at[pl.ds(base, chunk)], ceb[b], sce.at[b])
            pltpu.async_copy(dh.at[is_lo[b]], dhb[b].at[pl.ds(0, h2)],
                             sdh.at[b])
            pltpu.async_copy(dh.at[is_hi[b]], dhb[b].at[pl.ds(h2, h2)],
                             sdh.at[b])
            pltpu.async_copy(eh.at[id_lo[b]], ehb[b].at[pl.ds(0, h2)],
                             seh.at[b])
            pltpu.async_copy(eh.at[id_hi[b]], ehb[b].at[pl.ds(h2, h2)],
                             seh.at[b])

        def in_wait(b):
            pltpu.make_async_copy(ce.at[pl.ds(0, chunk)], ceb[b],
                                  sce.at[b]).wait()
            pltpu.make_async_copy(dh.at[is_lo[b]], dhb[b].at[pl.ds(0, h2)],
                                  sdh.at[b]).wait()
            pltpu.make_async_copy(dh.at[is_hi[b]], dhb[b].at[pl.ds(h2, h2)],
                                  sdh.at[b]).wait()
            pltpu.make_async_copy(eh.at[id_lo[b]], ehb[b].at[pl.ds(0, h2)],
                                  seh.at[b]).wait()
            pltpu.make_async_copy(eh.at[id_hi[b]], ehb[b].at[pl.ds(h2, h2)],
                                  seh.at[b]).wait()

        def out_issue(k, b):
            base = (k * NW + wid) * chunk
            pltpu.async_copy(ceb[b], enew_o.at[pl.ds(base, chunk)], soe.at[b])
            pltpu.async_copy(dhb[b].at[pl.ds(0, h2)], acc.at[ic_lo[b]],
                             sos.at[b], add=True)
            pltpu.async_copy(dhb[b].at[pl.ds(h2, h2)], acc.at[ic_hi[b]],
                             sos.at[b], add=True)

        def out_wait(b):
            pltpu.make_async_copy(ceb[b], enew_o.at[pl.ds(0, chunk)],
                                  soe.at[b]).wait()
            pltpu.make_async_copy(dhb[b].at[pl.ds(0, h2)], acc.at[ic_lo[b]],
                                  sos.at[b]).wait()
            pltpu.make_async_copy(dhb[b].at[pl.ds(h2, h2)], acc.at[ic_hi[b]],
                                  sos.at[b]).wait()

        def idx_keep(b):
            for g in range(h2 // L):
                ic_lo[b][pl.ds(g * L, L)] = id_lo[b][pl.ds(g * L, L)]
                ic_hi[b][pl.ds(g * L, L)] = id_hi[b][pl.ds(g * L, L)]

        for j in range(NBUF):
            idx_issue(j, j)
        for j in range(NBUF - 1):
            idx_wait(j)
            in_issue(j, j)

        carry0 = tuple(jnp.zeros((L,), jnp.float32) for _ in range(2 * grp))

        def outer_body(kk, bn):
            for b in range(NBUF):
                k = NBUF * kk + b
                pb = (b + NBUF - 1) % NBUF
                live = k < n_my

                @pl.when(live)
                def _():
                    in_wait(b)
                    idx_keep(b)

                @pl.when(jnp.logical_and(k >= 1, k - 1 < n_my))
                def _():
                    out_wait(pb)

                @pl.when(k + NBUF - 1 < n_my)
                def _():
                    idx_wait(pb)
                    in_issue(k + NBUF - 1, pb)

                @pl.when(k + NBUF < n_my)
                def _():
                    idx_issue(k + NBUF, b)

                def row_body(r, lc):
                    out = list(lc)
                    for g in range(grp):
                        sl = pl.ds(g * L, L)
                        x = ceb[b][r, sl] + dhb[b][r, sl] + ehb[b][r, sl]
                        ceb[b][r, sl] = x
                        dhb[b][r, sl] = _sigmoid(x)
                        out[g] = out[g] + x
                        out[grp + g] = out[grp + g] + x * x
                    return tuple(out)

                local0 = tuple(jnp.zeros((L,), jnp.float32)
                               for _ in range(2 * grp))
                local = lax.fori_loop(0, chunk, row_body, local0, unroll=False)
                mask = jnp.where(live, 1.0, 0.0)
                bn = tuple(bn[i] + local[i] * mask for i in range(2 * grp))

                @pl.when(live)
                def _():
                    out_issue(k, b)
            return bn

        bn = lax.fori_loop(0, kmax // NBUF, outer_body, carry0, unroll=False)

        for g in range(grp):
            bns_b[pl.ds(g * L, L)] = bn[g]
            bnq_b[pl.ds(g * L, L)] = bn[grp + g]
        pltpu.sync_copy(bns_b, bns_o.at[pl.ds(wid * d, d)])
        pltpu.sync_copy(bnq_b, bnq_o.at[pl.ds(wid * d, d)])

        plsc.subcore_barrier()

        @pl.when(s < NS - 1)
        def _():
            _spmem_dump(acc, dhb[0], psum_o, c * n, row0, rps)

        @pl.when(s == NS - 1)
        def _():
            _spmem_dump(acc, dhb[0], psum_o, c * n, row0, rps_last)

    return pass1


def _make_pass2(n, e_cnt, d, chunk):
    nch = e_cnt // chunk
    assert nch * chunk == e_cnt and chunk % 8 == 0
    base_chunks = nch // NW
    extra = nch % NW
    rps, rps_last = _row_split(n)
    grp = d // L
    kmax = _trip(base_chunks, extra)
    mesh = plsc.VectorSubcoreMesh(core_axis_name="c", subcore_axis_name="s",
                                  num_cores=NC, num_subcores=NS)

    @functools.partial(
        pl.kernel,
        out_type=[
            jax.ShapeDtypeStruct((NC * n, d), jnp.float32),
        ],
        mesh=mesh,
        scratch_types=(
            [pltpu.VMEM_SHARED((n, d), jnp.float32)]
            + [pltpu.VMEM((chunk // 2,), jnp.int32)] * (2 * NBUF)
            + [pltpu.VMEM((chunk // 2,), jnp.int32)] * (2 * NBUF)
            + [pltpu.VMEM((chunk // 2,), jnp.int32)] * (2 * NBUF)
            + [pltpu.VMEM((chunk, d), jnp.float32)] * NBUF
            + [pltpu.VMEM((chunk, d), jnp.float32)] * NBUF
            + [pltpu.SemaphoreType.DMA((NBUF,))] * 5
        ),
    )
    def pass2(enew, eee, src, dst, psum_o, acc, *rest):
        h2 = chunk // 2
        is_lo, is_hi = rest[0:NBUF], rest[NBUF:2 * NBUF]
        id_lo, id_hi = rest[2 * NBUF:3 * NBUF], rest[3 * NBUF:4 * NBUF]
        ic_lo, ic_hi = rest[4 * NBUF:5 * NBUF], rest[5 * NBUF:6 * NBUF]
        enb = rest[6 * NBUF:7 * NBUF]
        gb = rest[7 * NBUF:8 * NBUF]
        sis, sid, sen, sg, sos = rest[8 * NBUF:]

        c = lax.axis_index("c")
        s = lax.axis_index("s")
        wid = s * NC + c

        _zero_fill(gb[0], chunk)
        row0 = s * rps

        @pl.when(s < NS - 1)
        def _():
            _spmem_zero(acc, gb[0], row0, rps)

        @pl.when(s == NS - 1)
        def _():
            _spmem_zero(acc, gb[0], row0, rps_last)

        plsc.subcore_barrier()

        n_my = base_chunks + jnp.where(wid < extra, 1, 0)

        def idx_issue(k, b):
            base = (k * NW + wid) * chunk
            pltpu.async_copy(src.at[pl.ds(base, h2)], is_lo[b], sis.at[b])
            pltpu.async_copy(src.at[pl.ds(base + h2, h2)], is_hi[b], sis.at[b])
            pltpu.async_copy(dst.at[pl.ds(base, h2)], id_lo[b], sid.at[b])
            pltpu.async_copy(dst.at[pl.ds(base + h2, h2)], id_hi[b], sid.at[b])

        def idx_wait(b):
            pltpu.make_async_copy(src.at[pl.ds(0, h2)], is_lo[b],
                                  sis.at[b]).wait()
            pltpu.make_async_copy(src.at[pl.ds(0, h2)], is_hi[b],
                                  sis.at[b]).wait()
            pltpu.make_async_copy(dst.at[pl.ds(0, h2)], id_lo[b],
                                  sid.at[b]).wait()
            pltpu.make_async_copy(dst.at[pl.ds(0, h2)], id_hi[b],
                                  sid.at[b]).wait()

        def in_issue(k, b):
            base = (k * NW + wid) * chunk
            pltpu.async_copy(enew.at[pl.ds(base, chunk)], enb[b], sen.at[b])
            pltpu.async_copy(eee.at[is_lo[b]], gb[b].at[pl.ds(0, h2)],
                             sg.at[b])
            pltpu.async_copy(eee.at[is_hi[b]], gb[b].at[pl.ds(h2, h2)],
                             sg.at[b])

        def in_wait(b):
            pltpu.make_async_copy(enew.at[pl.ds(0, chunk)], enb[b],
                                  sen.at[b]).wait()
            pltpu.make_async_copy(eee.at[is_lo[b]], gb[b].at[pl.ds(0, h2)],
                                  sg.at[b]).wait()
            pltpu.make_async_copy(eee.at[is_hi[b]], gb[b].at[pl.ds(h2, h2)],
                                  sg.at[b]).wait()

        def out_issue(b):
            pltpu.async_copy(gb[b].at[pl.ds(0, h2)], acc.at[ic_lo[b]],
                             sos.at[b], add=True)
            pltpu.async_copy(gb[b].at[pl.ds(h2, h2)], acc.at[ic_hi[b]],
                             sos.at[b], add=True)

        def out_wait(b):
            pltpu.make_async_copy(gb[b].at[pl.ds(0, h2)], acc.at[ic_lo[b]],
                                  sos.at[b]).wait()
            pltpu.make_async_copy(gb[b].at[pl.ds(h2, h2)], acc.at[ic_hi[b]],
                                  sos.at[b]).wait()

        def idx_keep(b):
            for g in range(h2 // L):
                ic_lo[b][pl.ds(g * L, L)] = id_lo[b][pl.ds(g * L, L)]
                ic_hi[b][pl.ds(g * L, L)] = id_hi[b][pl.ds(g * L, L)]

        for j in range(NBUF):
            idx_issue(j, j)
        for j in range(NBUF - 1):
            idx_wait(j)
            in_issue(j, j)

        def outer_body(kk, carry):
            for b in range(NBUF):
                k = NBUF * kk + b
                pb = (b + NBUF - 1) % NBUF
                live = k < n_my

                @pl.when(live)
                def _():
                    in_wait(b)
                    idx_keep(b)

                @pl.when(jnp.logical_and(k >= 1, k - 1 < n_my))
                def _():
                    out_wait(pb)

                @pl.when(k + NBUF - 1 < n_my)
                def _():
                    idx_wait(pb)
                    in_issue(k + NBUF - 1, pb)

                @pl.when(k + NBUF < n_my)
                def _():
                    idx_issue(k + NBUF, b)

                def row_body(r, rc):
                    for g in range(grp):
                        sl = pl.ds(g * L, L)
                        sg_v = _sigmoid(enb[b][r, sl])
                        gb[b][r, sl] = gb[b][r, sl] * sg_v
                    return rc

                lax.fori_loop(0, chunk, row_body, 0, unroll=False)

                @pl.when(live)
                def _():
                    out_issue(b)
            return carry

        lax.fori_loop(0, kmax // NBUF, outer_body, 0, unroll=False)

        plsc.subcore_barrier()

        @pl.when(s < NS - 1)
        def _():
            _spmem_dump(acc, gb[0], psum_o, c * n, row0, rps)

        @pl.when(s == NS - 1)
        def _():
            _spmem_dump(acc, gb[0], psum_o, c * n, row0, rps_last)

    return pass2


def kernel(h, e, edge_index, A_w, A_b, B_w, B_b, C_w, C_b, D_w, D_b, E_w, E_b,
           bn_h_gamma, bn_h_beta, bn_e_gamma, bn_e_beta):
    n, d = h.shape
    e_cnt = e.shape[0]
    src = edge_index[0].astype(jnp.int32)
    dst = edge_index[1].astype(jnp.int32)

    dh, eh = pl.pallas_call(
        _node_mm2_body,
        out_shape=[jax.ShapeDtypeStruct((n, d), jnp.float32)] * 2,
    )(h, D_w, D_b.reshape(1, d), E_w, E_b.reshape(1, d))
    ah, bh = pl.pallas_call(
        _node_mm2_body,
        out_shape=[jax.ShapeDtypeStruct((n, d), jnp.float32)] * 2,
    )(h, A_w, A_b.reshape(1, d), B_w, B_b.reshape(1, d))

    br = 4000
    ce = pl.pallas_call(
        _edge_mm_body,
        grid=(e_cnt // br,),
        in_specs=[
            pl.BlockSpec((br, d), lambda i: (i, 0)),
            pl.BlockSpec((d, d), lambda i: (0, 0)),
            pl.BlockSpec((1, d), lambda i: (0, 0)),
        ],
        out_specs=pl.BlockSpec((br, d), lambda i: (i, 0)),
        out_shape=jax.ShapeDtypeStruct((e_cnt, d), jnp.float32),
    )(e, C_w, C_b.reshape(1, d))

    enew, psum, bns, bnq = _make_pass1(n, e_cnt, d, 64)(ce, dh, eh, src, dst)

    eee, esc, esh = pl.pallas_call(
        functools.partial(_stage_b_body, n_edges=float(e_cnt)),
        out_shape=[
            jax.ShapeDtypeStruct((n, d), jnp.float32),
            jax.ShapeDtypeStruct((8, d), jnp.float32),
            jax.ShapeDtypeStruct((8, d), jnp.float32),
        ],
    )(psum, bh, bns.reshape(NW, d), bnq.reshape(NW, d),
      bn_e_gamma.reshape(1, d), bn_e_beta.reshape(1, d))

    br2 = 4000
    e_out = pl.pallas_call(
        _eout_body,
        grid=(e_cnt // br2,),
        in_specs=[
            pl.BlockSpec((br2, d), lambda i: (i, 0)),
            pl.BlockSpec((br2, d), lambda i: (i, 0)),
            pl.BlockSpec((1, d), lambda i: (0, 0)),
            pl.BlockSpec((1, d), lambda i: (0, 0)),
        ],
        out_specs=pl.BlockSpec((br2, d), lambda i: (i, 0)),
        out_shape=jax.ShapeDtypeStruct((e_cnt, d), jnp.float32),
    )(enew, e, esc[0].reshape(1, d), esh[0].reshape(1, d))

    (psumh,) = _make_pass2(n, e_cnt, d, 64)(enew, eee, src, dst)

    h_out = pl.pallas_call(
        _stage_c_body,
        out_shape=jax.ShapeDtypeStruct((n, d), jnp.float32),
    )(ah, psumh, h, bn_h_gamma.reshape(1, d), bn_h_beta.reshape(1, d))

    return (h_out, e_out)
